# baseline (speedup 1.0000x reference)
import functools

import jax
import jax.numpy as jnp
from jax import lax
from jax.experimental import pallas as pl
from jax.experimental.pallas import tpu as pltpu
from jax.experimental.pallas import tpu_sc as plsc

NC, NS = 2, 16
NW = NC * NS
GCH = 128
TD = 256


def _sc_gather(table, idx, bn_pad):
    b_per_w = bn_pad // NW
    mesh = plsc.VectorSubcoreMesh(core_axis_name="c", subcore_axis_name="s")

    @functools.partial(
        pl.kernel,
        out_type=jax.ShapeDtypeStruct((bn_pad, TD), jnp.float32),
        mesh=mesh,
        scratch_types=[
            pltpu.VMEM((b_per_w,), jnp.int32),
            pltpu.VMEM((GCH, TD), jnp.float32),
            pltpu.VMEM((GCH, TD), jnp.float32),
            pltpu.SemaphoreType.DMA,
            pltpu.SemaphoreType.DMA,
        ],
    )
    def gather_kernel(t_hbm, idx_hbm, g_hbm, idx_v, r0, r1, s0, s1):
        wid = lax.axis_index("s") * NC + lax.axis_index("c")
        base = wid * b_per_w
        pltpu.sync_copy(idx_hbm.at[pl.ds(base, b_per_w)], idx_v)

        nch = b_per_w // (2 * GCH)

        @pl.loop(0, nch)
        def _(i):
            off = i * (2 * GCH)
            cp0 = pltpu.async_copy(t_hbm.at[idx_v.at[pl.ds(off, GCH)]], r0, s0)
            cp1 = pltpu.async_copy(
                t_hbm.at[idx_v.at[pl.ds(off + GCH, GCH)]], r1, s1)
            cp0.wait()
            pltpu.sync_copy(r0, g_hbm.at[pl.ds(base + off, GCH)])
            cp1.wait()
            pltpu.sync_copy(r1, g_hbm.at[pl.ds(base + off + GCH, GCH)])

    return gather_kernel(table, idx)


def _table_body(p0, p1, fl, o_ref):
    ii = lax.broadcasted_iota(jnp.int32, (64, 64), 0)
    jj = lax.broadcasted_iota(jnp.int32, (64, 64), 1)
    eye = (ii == jj).astype(jnp.float32)
    dn = (((0,), (0,)), ((), ()))
    tr = lambda m: lax.dot_general(m[0], eye, dn,
                                   preferred_element_type=jnp.float32)
    o_ref[:, 0:64] = tr(p0)
    o_ref[:, 64:128] = tr(p1)
    o_ref[:, 128:192] = tr(fl)


def _build_table(pc0, pc1, fl, B, C, HW):
    T = 4096
    map_spec = pl.BlockSpec((1, C, T), lambda b, j: (b, 0, j))
    return pl.pallas_call(
        _table_body,
        out_shape=jax.ShapeDtypeStruct((B * HW, TD), jnp.float32),
        grid=(B, HW // T),
        in_specs=[map_spec, map_spec, map_spec],
        out_specs=pl.BlockSpec((T, TD), lambda b, j: (b * (HW // T) + j, 0)),
        compiler_params=pltpu.CompilerParams(
            dimension_semantics=("parallel", "parallel")),
    )(pc0.reshape(B, C, HW), pc1.reshape(B, C, HW), fl.reshape(B, C, HW))


def _gelu_exact(x):
    return 0.5 * x * (1.0 + lax.erf(x * 0.7071067811865476))


def _mlp_body(g, wl0, wl1, bl, w1a, w1b, b1r, w2r, b2r, w3r, b3r,
              w4r, b4r, o_ref):
    f32 = jnp.float32
    gb = g[...]
    g0, g1, g2 = gb[:, 0:64], gb[:, 64:128], gb[:, 128:192]
    x = (jnp.dot(g0, wl0[...], preferred_element_type=f32)
         + jnp.dot(g1, wl1[...], preferred_element_type=f32) + bl[...])
    h = (jnp.dot(x, w1a[...], preferred_element_type=f32)
         + jnp.dot(g2, w1b[...], preferred_element_type=f32) + b1r[...])
    h = _gelu_exact(h)
    h = _gelu_exact(jnp.dot(h, w2r[...], preferred_element_type=f32)
                    + b2r[...])
    h = _gelu_exact(jnp.dot(h, w3r[...], preferred_element_type=f32)
                    + b3r[...])
    s = jnp.dot(h, w4r[...], preferred_element_type=f32)[:, 0] + b4r[0]
    o_ref[...] = jax.nn.sigmoid(s)


def kernel(pc0_map, pc1_map, flow_map, lidar_voxel_coords, radar_voxel_coords,
           W_lin, b_lin, W1, b1, W2, b2, W3, b3, W4, b4):
    B, C, H, W = pc0_map.shape
    NL = lidar_voxel_coords.shape[1]
    NR = radar_voxel_coords.shape[1]
    N = NL + NR
    BN = B * N
    BN_pad = -(-BN // (NW * 2 * GCH)) * (NW * 2 * GCH)

    coords = jnp.concatenate([lidar_voxel_coords, radar_voxel_coords], axis=1)
    idx = (coords[..., 1] * W + coords[..., 2]
           + jnp.arange(B, dtype=jnp.int32)[:, None] * (H * W))
    idx = jnp.pad(idx.reshape(BN), (0, BN_pad - BN)).astype(jnp.int32)

    table = _build_table(pc0_map, pc1_map, flow_map, B, C, H * W)

    g = _sc_gather(table, idx, BN_pad)

    BLK = 512
    full = lambda shape: pl.BlockSpec(shape, lambda i: tuple(0 for _ in shape))
    out = pl.pallas_call(
        _mlp_body,
        out_shape=jax.ShapeDtypeStruct((BN,), jnp.float32),
        grid=(BN // BLK,),
        in_specs=[pl.BlockSpec((BLK, TD), lambda i: (i, 0)),
                  full((64, 64)), full((64, 64)), full((64,)),
                  full((64, 64)), full((64, 64)), full((64,)),
                  full((64, 32)), full((32,)),
                  full((32, 16)), full((16,)),
                  full((16, 1)), full((1,))],
        out_specs=pl.BlockSpec((BLK,), lambda i: (i,)),
        compiler_params=pltpu.CompilerParams(
            dimension_semantics=("parallel",)),
    )(g,
      W_lin[:64], W_lin[64:], b_lin,
      W1[:64], W1[64:], b1,
      W2, b2, W3, b3, W4, b4)

    return out.reshape(B, N)


# speedup vs baseline: 1.5833x; 1.5833x over previous
import functools

import jax
import jax.numpy as jnp
from jax import lax
from jax.experimental import pallas as pl
from jax.experimental.pallas import tpu as pltpu
from jax.experimental.pallas import tpu_sc as plsc

NC, NS = 2, 16
NW = NC * NS
GCH = 128
TD = 256


def _sc_gather(table, idx, bn_pad):
    b_per_w = bn_pad // NW
    mesh = plsc.VectorSubcoreMesh(core_axis_name="c", subcore_axis_name="s")

    @functools.partial(
        pl.kernel,
        out_type=jax.ShapeDtypeStruct((bn_pad, TD), jnp.float32),
        mesh=mesh,
        scratch_types=[
            pltpu.VMEM((b_per_w,), jnp.int32),
            pltpu.VMEM((GCH, TD), jnp.float32),
            pltpu.VMEM((GCH, TD), jnp.float32),
            pltpu.SemaphoreType.DMA,
            pltpu.SemaphoreType.DMA,
        ],
    )
    def gather_kernel(t_hbm, idx_hbm, g_hbm, idx_v, r0, r1, s0, s1):
        wid = lax.axis_index("s") * NC + lax.axis_index("c")
        base = wid * b_per_w
        pltpu.sync_copy(idx_hbm.at[pl.ds(base, b_per_w)], idx_v)

        nch = b_per_w // (2 * GCH)

        @pl.loop(0, nch)
        def _(i):
            off = i * (2 * GCH)
            cp0 = pltpu.async_copy(t_hbm.at[idx_v.at[pl.ds(off, GCH)]], r0, s0)
            cp1 = pltpu.async_copy(
                t_hbm.at[idx_v.at[pl.ds(off + GCH, GCH)]], r1, s1)
            cp0.wait()
            pltpu.sync_copy(r0, g_hbm.at[pl.ds(base + off, GCH)])
            cp1.wait()
            pltpu.sync_copy(r1, g_hbm.at[pl.ds(base + off + GCH, GCH)])

    return gather_kernel(table, idx)


def _table_body(p0, p1, fl, o_ref):
    ii = lax.broadcasted_iota(jnp.int32, (64, 64), 0)
    jj = lax.broadcasted_iota(jnp.int32, (64, 64), 1)
    eye = (ii == jj).astype(jnp.float32)
    dn = (((0,), (0,)), ((), ()))

    def tr(m):
        x = jnp.reshape(m[0], (64, m.shape[2] * m.shape[3]))
        return lax.dot_general(x, eye, dn, preferred_element_type=jnp.float32)

    o_ref[:, 0:64] = tr(p0)
    o_ref[:, 64:128] = tr(p1)
    o_ref[:, 128:192] = tr(fl)


def _build_table(pc0, pc1, fl, B, C, HW):
    HB = 8
    T = HB * 512
    map_spec = pl.BlockSpec((1, C, HB, 512), lambda b, j: (b, 0, j, 0))
    return pl.pallas_call(
        _table_body,
        out_shape=jax.ShapeDtypeStruct((B * HW, TD), jnp.float32),
        grid=(B, HW // T),
        in_specs=[map_spec, map_spec, map_spec],
        out_specs=pl.BlockSpec((T, TD), lambda b, j: (b * (HW // T) + j, 0)),
        compiler_params=pltpu.CompilerParams(
            dimension_semantics=("parallel", "parallel")),
    )(pc0, pc1, fl)


def _gelu_exact(x):
    return 0.5 * x * (1.0 + lax.erf(x * 0.7071067811865476))


def _mlp_body(g, wl0, wl1, bl, w1a, w1b, b1r, w2r, b2r, w3r, b3r,
              w4r, b4r, o_ref):
    f32 = jnp.float32
    gb = g[...]
    g0, g1, g2 = gb[:, 0:64], gb[:, 64:128], gb[:, 128:192]
    x = (jnp.dot(g0, wl0[...], preferred_element_type=f32)
         + jnp.dot(g1, wl1[...], preferred_element_type=f32) + bl[...])
    h = (jnp.dot(x, w1a[...], preferred_element_type=f32)
         + jnp.dot(g2, w1b[...], preferred_element_type=f32) + b1r[...])
    h = _gelu_exact(h)
    h = _gelu_exact(jnp.dot(h, w2r[...], preferred_element_type=f32)
                    + b2r[...])
    h = _gelu_exact(jnp.dot(h, w3r[...], preferred_element_type=f32)
                    + b3r[...])
    s = jnp.dot(h, w4r[...], preferred_element_type=f32)[:, 0] + b4r[0]
    o_ref[...] = jax.nn.sigmoid(s)


def kernel(pc0_map, pc1_map, flow_map, lidar_voxel_coords, radar_voxel_coords,
           W_lin, b_lin, W1, b1, W2, b2, W3, b3, W4, b4):
    B, C, H, W = pc0_map.shape
    NL = lidar_voxel_coords.shape[1]
    NR = radar_voxel_coords.shape[1]
    N = NL + NR
    BN = B * N
    BN_pad = -(-BN // (NW * 2 * GCH)) * (NW * 2 * GCH)

    coords = jnp.concatenate([lidar_voxel_coords, radar_voxel_coords], axis=1)
    idx = (coords[..., 1] * W + coords[..., 2]
           + jnp.arange(B, dtype=jnp.int32)[:, None] * (H * W))
    idx = jnp.pad(idx.reshape(BN), (0, BN_pad - BN)).astype(jnp.int32)

    table = _build_table(pc0_map, pc1_map, flow_map, B, C, H * W)

    g = _sc_gather(table, idx, BN_pad)

    BLK = 512
    full = lambda shape: pl.BlockSpec(shape, lambda i: tuple(0 for _ in shape))
    out = pl.pallas_call(
        _mlp_body,
        out_shape=jax.ShapeDtypeStruct((BN,), jnp.float32),
        grid=(BN // BLK,),
        in_specs=[pl.BlockSpec((BLK, TD), lambda i: (i, 0)),
                  full((64, 64)), full((64, 64)), full((64,)),
                  full((64, 64)), full((64, 64)), full((64,)),
                  full((64, 32)), full((32,)),
                  full((32, 16)), full((16,)),
                  full((16, 1)), full((1,))],
        out_specs=pl.BlockSpec((BLK,), lambda i: (i,)),
        compiler_params=pltpu.CompilerParams(
            dimension_semantics=("parallel",)),
    )(g,
      W_lin[:64], W_lin[64:], b_lin,
      W1[:64], W1[64:], b1,
      W2, b2, W3, b3, W4, b4)

    return out.reshape(B, N)


# speedup vs baseline: 1.8221x; 1.1508x over previous
import functools

import jax
import jax.numpy as jnp
from jax import lax
from jax.experimental import pallas as pl
from jax.experimental.pallas import tpu as pltpu
from jax.experimental.pallas import tpu_sc as plsc

NC, NS = 2, 16
NW = NC * NS
GCH = 128
TD = 256


def _sc_gather(table, idx, bn_pad):
    b_per_w = bn_pad // NW
    mesh = plsc.VectorSubcoreMesh(core_axis_name="c", subcore_axis_name="s")

    @functools.partial(
        pl.kernel,
        out_type=jax.ShapeDtypeStruct((bn_pad, TD), jnp.float32),
        mesh=mesh,
        scratch_types=[
            pltpu.VMEM((b_per_w,), jnp.int32),
            pltpu.VMEM((GCH, TD), jnp.float32),
            pltpu.VMEM((GCH, TD), jnp.float32),
            pltpu.SemaphoreType.DMA,
            pltpu.SemaphoreType.DMA,
        ],
    )
    def gather_kernel(t_hbm, idx_hbm, g_hbm, idx_v, r0, r1, s0, s1):
        wid = lax.axis_index("s") * NC + lax.axis_index("c")
        base = wid * b_per_w
        pltpu.sync_copy(idx_hbm.at[pl.ds(base, b_per_w)], idx_v)

        nch = b_per_w // (2 * GCH)

        @pl.loop(0, nch)
        def _(i):
            off = i * (2 * GCH)
            cp0 = pltpu.async_copy(t_hbm.at[idx_v.at[pl.ds(off, GCH)]], r0, s0)
            cp1 = pltpu.async_copy(
                t_hbm.at[idx_v.at[pl.ds(off + GCH, GCH)]], r1, s1)
            cp0.wait()
            pltpu.sync_copy(r0, g_hbm.at[pl.ds(base + off, GCH)])
            cp1.wait()
            pltpu.sync_copy(r1, g_hbm.at[pl.ds(base + off + GCH, GCH)])

    return gather_kernel(table, idx)


def _table_body(p0, p1, fl, o_ref):
    ii = lax.broadcasted_iota(jnp.int32, (64, 64), 0)
    jj = lax.broadcasted_iota(jnp.int32, (64, 64), 1)
    eye = (ii == jj).astype(jnp.float32)
    dn = (((0,), (0,)), ((), ()))

    def tr(m):
        x = jnp.reshape(m[0], (64, m.shape[2] * m.shape[3]))
        return lax.dot_general(x, eye, dn, preferred_element_type=jnp.float32)

    o_ref[:, 0:64] = tr(p0)
    o_ref[:, 64:128] = tr(p1)
    o_ref[:, 128:192] = tr(fl)


def _build_table(pc0, pc1, fl, b, C, HW):
    HB = 8
    T = HB * 512
    map_spec = pl.BlockSpec((1, C, HB, 512), lambda j: (b, 0, j, 0))
    return pl.pallas_call(
        _table_body,
        out_shape=jax.ShapeDtypeStruct((HW, TD), jnp.float32),
        grid=(HW // T,),
        in_specs=[map_spec, map_spec, map_spec],
        out_specs=pl.BlockSpec((T, TD), lambda j: (j, 0)),
        compiler_params=pltpu.CompilerParams(
            dimension_semantics=("parallel",)),
    )(pc0, pc1, fl)


def _gelu_exact(x):
    return 0.5 * x * (1.0 + lax.erf(x * 0.7071067811865476))


def _mlp_body(g, wl0, wl1, bl, w1a, w1b, b1r, w2r, b2r, w3r, b3r,
              w4r, b4r, o_ref):
    f32 = jnp.float32
    gb = g[...]
    g0, g1, g2 = gb[:, 0:64], gb[:, 64:128], gb[:, 128:192]
    x = (jnp.dot(g0, wl0[...], preferred_element_type=f32)
         + jnp.dot(g1, wl1[...], preferred_element_type=f32) + bl[...])
    h = (jnp.dot(x, w1a[...], preferred_element_type=f32)
         + jnp.dot(g2, w1b[...], preferred_element_type=f32) + b1r[...])
    h = _gelu_exact(h)
    h = _gelu_exact(jnp.dot(h, w2r[...], preferred_element_type=f32)
                    + b2r[...])
    h = _gelu_exact(jnp.dot(h, w3r[...], preferred_element_type=f32)
                    + b3r[...])
    s = jnp.dot(h, w4r[...], preferred_element_type=f32)[:, 0] + b4r[0]
    o_ref[...] = jax.nn.sigmoid(s)


def _mlp(g, N, weights):
    BLK = 512
    full = lambda shape: pl.BlockSpec(shape, lambda i: tuple(0 for _ in shape))
    return pl.pallas_call(
        _mlp_body,
        out_shape=jax.ShapeDtypeStruct((N,), jnp.float32),
        grid=(N // BLK,),
        in_specs=[pl.BlockSpec((BLK, TD), lambda i: (i, 0)),
                  full((64, 64)), full((64, 64)), full((64,)),
                  full((64, 64)), full((64, 64)), full((64,)),
                  full((64, 32)), full((32,)),
                  full((32, 16)), full((16,)),
                  full((16, 1)), full((1,))],
        out_specs=pl.BlockSpec((BLK,), lambda i: (i,)),
        compiler_params=pltpu.CompilerParams(
            dimension_semantics=("parallel",)),
    )(g, *weights)


def kernel(pc0_map, pc1_map, flow_map, lidar_voxel_coords, radar_voxel_coords,
           W_lin, b_lin, W1, b1, W2, b2, W3, b3, W4, b4):
    B, C, H, W = pc0_map.shape
    NL = lidar_voxel_coords.shape[1]
    NR = radar_voxel_coords.shape[1]
    N = NL + NR
    N_pad = -(-N // (NW * 2 * GCH)) * (NW * 2 * GCH)

    coords = jnp.concatenate([lidar_voxel_coords, radar_voxel_coords], axis=1)
    idx = (coords[..., 1] * W + coords[..., 2]).astype(jnp.int32)
    idx = jnp.pad(idx, ((0, 0), (0, N_pad - N)))

    weights = (W_lin[:64], W_lin[64:], b_lin, W1[:64], W1[64:], b1,
               W2, b2, W3, b3, W4, b4)

    outs = []
    for b in range(B):
        table = _build_table(pc0_map, pc1_map, flow_map, b, C, H * W)
        g = _sc_gather(table, idx[b], N_pad)
        outs.append(_mlp(g, N, weights))
    return jnp.stack(outs, axis=0)
